# baseline (speedup 1.0000x reference)
import functools

import jax
import jax.numpy as jnp
from jax import lax
from jax.experimental import pallas as pl
from jax.experimental.pallas import tpu as pltpu
from jax.experimental.pallas import tpu_sc as plsc

N = 10000
E = 160000
H = 32
N_ATOM = 16
ITERS = 2

GW = 128
EPAD = 163840
BE = 1024

_NC = 2
_NS = 16


@functools.lru_cache(maxsize=1)
def _vector_mesh():
  return plsc.VectorSubcoreMesh(
      core_axis_name="core", subcore_axis_name="subcore",
      num_cores=_NC, num_subcores=_NS)


def _sc_gather(table, idx2d):
  @functools.partial(
      pl.kernel,
      out_type=jax.ShapeDtypeStruct((EPAD, H), jnp.float32),
      mesh=_vector_mesh(),
      compiler_params=pltpu.CompilerParams(use_tc_tiling_on_sc=False),
  )
  def gk(tab_hbm, i_hbm, o_hbm):
    def body(i_vmem, o_vmem):
      pltpu.sync_copy(tab_hbm.at[i_vmem.at[0]], o_vmem)

    pltpu.emit_pipeline(
        body,
        grid=(EPAD // GW,),
        in_specs=[pl.BlockSpec((1, GW), lambda i: (0, i))],
        out_specs=[pl.BlockSpec((GW, H), lambda i: (i, 0))],
        core_axis_name=("core", "subcore"),
        dimension_semantics=(pltpu.PARALLEL,),
    )(i_hbm, o_hbm)

  return gk(table, idx2d)


def _sc_scatter_add(msg, idx2d, zeros_tab):
  @functools.partial(
      pl.kernel,
      out_type=jax.ShapeDtypeStruct((_NC, N, H), jnp.float32),
      mesh=_vector_mesh(),
      scratch_types=[pltpu.VMEM_SHARED((N, H), jnp.float32)],
      compiler_params=pltpu.CompilerParams(use_tc_tiling_on_sc=False),
  )
  def sk(m_hbm, i_hbm, z_hbm, o_hbm, acc_shared):
    cid = lax.axis_index("core")
    sid = lax.axis_index("subcore")

    @pl.when(sid == 0)
    def _():
      pltpu.sync_copy(z_hbm, acc_shared)

    plsc.subcore_barrier()

    def body(m_vmem, i_vmem):
      pltpu.sync_copy(m_vmem, acc_shared.at[i_vmem.at[0]], add=True)

    pltpu.emit_pipeline(
        body,
        grid=(EPAD // GW,),
        in_specs=[
            pl.BlockSpec((GW, H), lambda i: (i, 0)),
            pl.BlockSpec((1, GW), lambda i: (0, i)),
        ],
        out_specs=[],
        core_axis_name=("core", "subcore"),
        dimension_semantics=(pltpu.PARALLEL,),
    )(m_hbm, i_hbm)

    plsc.subcore_barrier()
    rows = N // _NS
    sl = pl.ds(sid * rows, rows)
    pltpu.sync_copy(acc_shared.at[sl], o_hbm.at[cid].at[sl])

  return sk(msg, idx2d, zeros_tab)


def _msg_body(ea_ref, xs_ref, w1_ref, b1_ref, w2_ref, b2_ref, o_ref):
  eh = jnp.maximum(
      jnp.dot(ea_ref[...], w1_ref[...], preferred_element_type=jnp.float32)
      + b1_ref[...], 0.0)
  we = jnp.dot(eh, w2_ref[...], preferred_element_type=jnp.float32) + b2_ref[...]
  xs = xs_ref[...]
  acc = xs[:, 0:1] * we[:, 0:H]
  for hh in range(1, H):
    acc = acc + xs[:, hh:hh + 1] * we[:, hh * H:(hh + 1) * H]
  blk = pl.program_id(0)
  row = blk * BE + lax.broadcasted_iota(jnp.int32, (BE, H), 0)
  o_ref[...] = jnp.where(row < E, acc, 0.0)


def _tc_msg(ea_pad, xs, W1, b1r, W2, b2r):
  return pl.pallas_call(
      _msg_body,
      grid=(EPAD // BE,),
      in_specs=[
          pl.BlockSpec((BE, 4), lambda i: (i, 0)),
          pl.BlockSpec((BE, H), lambda i: (i, 0)),
          pl.BlockSpec((4, 128), lambda i: (0, 0)),
          pl.BlockSpec((1, 128), lambda i: (0, 0)),
          pl.BlockSpec((128, H * H), lambda i: (0, 0)),
          pl.BlockSpec((1, H * H), lambda i: (0, 0)),
      ],
      out_specs=pl.BlockSpec((BE, H), lambda i: (i, 0)),
      out_shape=jax.ShapeDtypeStruct((EPAD, H), jnp.float32),
  )(ea_pad, xs, W1, b1r, W2, b2r)


def _node_body(agg2_ref, out_ref, h_ref, wr_ref, bc_ref, wih_ref, whh_ref,
               bih_ref, bhh_ref, o_ref):
  agg = agg2_ref[0] + agg2_ref[1]
  out = out_ref[...]
  h = h_ref[...]
  m = jnp.maximum(
      agg + jnp.dot(out, wr_ref[...], preferred_element_type=jnp.float32)
      + bc_ref[...], 0.0)
  gi = jnp.dot(m, wih_ref[...], preferred_element_type=jnp.float32) + bih_ref[...]
  gh = jnp.dot(h, whh_ref[...], preferred_element_type=jnp.float32) + bhh_ref[...]
  r = jax.nn.sigmoid(gi[:, :H] + gh[:, :H])
  z = jax.nn.sigmoid(gi[:, H:2 * H] + gh[:, H:2 * H])
  n = jnp.tanh(gi[:, 2 * H:] + r * gh[:, 2 * H:])
  o_ref[...] = (1.0 - z) * n + z * h


def _tc_node_update(agg2, out, h, W_root, bc, W_ihT, W_hhT, bih, bhh):
  return pl.pallas_call(
      _node_body,
      out_shape=jax.ShapeDtypeStruct((N, H), jnp.float32),
  )(agg2, out, h, W_root, bc, W_ihT, W_hhT, bih, bhh)


def _emb_body(x_ref, emb_ref, o_ref):
  a = x_ref[...]
  oh = (a == lax.broadcasted_iota(jnp.int32, (N, N_ATOM), 1)).astype(jnp.float32)
  o_ref[...] = jnp.maximum(
      jnp.dot(oh, emb_ref[...], preferred_element_type=jnp.float32), 0.0)


def _tc_embed(x, emb):
  return pl.pallas_call(
      _emb_body,
      out_shape=jax.ShapeDtypeStruct((N, H), jnp.float32),
  )(x, emb)


def _final_body(h_ref, x_ref, wo1_ref, bo1_ref, wo2_ref, bo2_ref, o_ref):
  h = h_ref[...]
  t = jnp.maximum(
      jnp.dot(h, wo1_ref[...], preferred_element_type=jnp.float32)
      + bo1_ref[...], 0.0)
  o = jnp.dot(t, wo2_ref[...], preferred_element_type=jnp.float32) + bo2_ref[...]
  a = x_ref[...]
  oh = (a == lax.broadcasted_iota(jnp.int32, (N, N_ATOM), 1)).astype(jnp.float32)
  feat = jnp.concatenate([o, oh], axis=1)
  nrm = jnp.sqrt(jnp.sum(feat * feat, axis=1, keepdims=True))
  o_ref[...] = feat / jnp.maximum(nrm, 1e-12)


def _tc_final(h, x, Wo1, bo1, Wo2, bo2):
  return pl.pallas_call(
      _final_body,
      out_shape=jax.ShapeDtypeStruct((N, H + N_ATOM), jnp.float32),
  )(h, x, Wo1, bo1, Wo2, bo2)


def kernel(x, edge_index, edge_attr, emb, W1, b1, W2, b2, W_root, b_conv,
           W_ih, W_hh, b_ih, b_hh, Wo1, bo1, Wo2, bo2):
  src = edge_index[0]
  dst = edge_index[1]
  pad = EPAD - E
  src2 = jnp.pad(src, (0, pad)).reshape(1, EPAD)
  dst2 = jnp.pad(dst, (0, pad)).reshape(1, EPAD)
  ea_pad = jnp.pad(edge_attr, ((0, pad), (0, 0)))
  zeros_tab = jnp.zeros((N, H), jnp.float32)

  b1r = b1.reshape(1, 128)
  b2r = b2.reshape(1, H * H)
  bcr = b_conv.reshape(1, H)
  bihr = b_ih.reshape(1, 3 * H)
  bhhr = b_hh.reshape(1, 3 * H)
  bo1r = bo1.reshape(1, H)
  bo2r = bo2.reshape(1, H)
  W_ihT = W_ih.T
  W_hhT = W_hh.T

  out = _tc_embed(x, emb)
  h = out
  for _ in range(ITERS):
    xs = _sc_gather(out, src2)
    msg = _tc_msg(ea_pad, xs, W1, b1r, W2, b2r)
    agg2 = _sc_scatter_add(msg, dst2, zeros_tab)
    h = _tc_node_update(agg2, out, h, W_root, bcr, W_ihT, W_hhT, bihr, bhhr)
    out = h
  return _tc_final(h, x, Wo1, bo1r, Wo2, bo2r)


# speedup vs baseline: 3.0269x; 3.0269x over previous
import functools

import jax
import jax.numpy as jnp
from jax import lax
from jax.experimental import pallas as pl
from jax.experimental.pallas import tpu as pltpu
from jax.experimental.pallas import tpu_sc as plsc

N = 10000
E = 160000
H = 32
N_ATOM = 16
ITERS = 2

GW = 128
EPAD = 163840
BE = 1024

_NC = 2
_NS = 16


@functools.lru_cache(maxsize=1)
def _vector_mesh():
  return plsc.VectorSubcoreMesh(
      core_axis_name="core", subcore_axis_name="subcore",
      num_cores=_NC, num_subcores=_NS)


def _sc_gather(table, idx2d):
  @functools.partial(
      pl.kernel,
      out_type=jax.ShapeDtypeStruct((EPAD, H), jnp.float32),
      mesh=_vector_mesh(),
      compiler_params=pltpu.CompilerParams(use_tc_tiling_on_sc=False),
  )
  def gk(tab_hbm, i_hbm, o_hbm):
    def body(i_vmem, o_vmem):
      pltpu.sync_copy(tab_hbm.at[i_vmem.at[0]], o_vmem)

    pltpu.emit_pipeline(
        body,
        grid=(EPAD // GW,),
        in_specs=[pl.BlockSpec((1, GW), lambda i: (0, i))],
        out_specs=[pl.BlockSpec((GW, H), lambda i: (i, 0))],
        core_axis_name=("core", "subcore"),
        dimension_semantics=(pltpu.PARALLEL,),
    )(i_hbm, o_hbm)

  return gk(table, idx2d)


def _sc_scatter_add(msg, idx2d, zeros_tab):
  @functools.partial(
      pl.kernel,
      out_type=jax.ShapeDtypeStruct((_NC, N, H), jnp.float32),
      mesh=_vector_mesh(),
      scratch_types=[pltpu.VMEM_SHARED((N, H), jnp.float32)],
      compiler_params=pltpu.CompilerParams(use_tc_tiling_on_sc=False),
  )
  def sk(m_hbm, i_hbm, z_hbm, o_hbm, acc_shared):
    cid = lax.axis_index("core")
    sid = lax.axis_index("subcore")

    @pl.when(sid == 0)
    def _():
      pltpu.sync_copy(z_hbm, acc_shared)

    plsc.subcore_barrier()

    def body(m_vmem, i_vmem):
      pltpu.sync_copy(m_vmem, acc_shared.at[i_vmem.at[0]], add=True)

    pltpu.emit_pipeline(
        body,
        grid=(EPAD // GW,),
        in_specs=[
            pl.BlockSpec((GW, H), lambda i: (i, 0)),
            pl.BlockSpec((1, GW), lambda i: (0, i)),
        ],
        out_specs=[],
        core_axis_name=("core", "subcore"),
        dimension_semantics=(pltpu.PARALLEL,),
    )(m_hbm, i_hbm)

    plsc.subcore_barrier()
    rows = N // _NS
    sl = pl.ds(sid * rows, rows)
    pltpu.sync_copy(acc_shared.at[sl], o_hbm.at[cid].at[sl])

  return sk(msg, idx2d, zeros_tab)


def _msg_body(ea_ref, xs_ref, w1_ref, b1_ref, w2o_ref, b2o_ref, rsum_ref,
              o_ref):
  eh = jnp.maximum(
      jnp.dot(ea_ref[...], w1_ref[...], preferred_element_type=jnp.float32)
      + b1_ref[...], 0.0)
  we = jnp.dot(eh, w2o_ref[...], preferred_element_type=jnp.float32) + b2o_ref[...]
  xsrep = pltpu.repeat(xs_ref[...], H, axis=1)
  acc = jnp.dot(we * xsrep, rsum_ref[...], preferred_element_type=jnp.float32)
  blk = pl.program_id(0)
  row = blk * BE + lax.broadcasted_iota(jnp.int32, (BE, H), 0)
  o_ref[...] = jnp.where(row < E, acc, 0.0)


def _tc_msg(ea_pad, xs, W1, b1r, W2o, b2o, rsum):
  return pl.pallas_call(
      _msg_body,
      grid=(EPAD // BE,),
      in_specs=[
          pl.BlockSpec((BE, 4), lambda i: (i, 0)),
          pl.BlockSpec((BE, H), lambda i: (i, 0)),
          pl.BlockSpec((4, 128), lambda i: (0, 0)),
          pl.BlockSpec((1, 128), lambda i: (0, 0)),
          pl.BlockSpec((128, H * H), lambda i: (0, 0)),
          pl.BlockSpec((1, H * H), lambda i: (0, 0)),
          pl.BlockSpec((H * H, H), lambda i: (0, 0)),
      ],
      out_specs=pl.BlockSpec((BE, H), lambda i: (i, 0)),
      out_shape=jax.ShapeDtypeStruct((EPAD, H), jnp.float32),
  )(ea_pad, xs, W1, b1r, W2o, b2o, rsum)


def _node_body(agg2_ref, out_ref, h_ref, wr_ref, bc_ref, wih_ref, whh_ref,
               bih_ref, bhh_ref, o_ref):
  agg = agg2_ref[0] + agg2_ref[1]
  out = out_ref[...]
  h = h_ref[...]
  m = jnp.maximum(
      agg + jnp.dot(out, wr_ref[...], preferred_element_type=jnp.float32)
      + bc_ref[...], 0.0)
  gi = jnp.dot(m, wih_ref[...], preferred_element_type=jnp.float32) + bih_ref[...]
  gh = jnp.dot(h, whh_ref[...], preferred_element_type=jnp.float32) + bhh_ref[...]
  r = jax.nn.sigmoid(gi[:, :H] + gh[:, :H])
  z = jax.nn.sigmoid(gi[:, H:2 * H] + gh[:, H:2 * H])
  n = jnp.tanh(gi[:, 2 * H:] + r * gh[:, 2 * H:])
  o_ref[...] = (1.0 - z) * n + z * h


def _tc_node_update(agg2, out, h, W_root, bc, W_ihT, W_hhT, bih, bhh):
  return pl.pallas_call(
      _node_body,
      out_shape=jax.ShapeDtypeStruct((N, H), jnp.float32),
  )(agg2, out, h, W_root, bc, W_ihT, W_hhT, bih, bhh)


def _emb_body(x_ref, emb_ref, o_ref):
  a = x_ref[...]
  oh = (a == lax.broadcasted_iota(jnp.int32, (N, N_ATOM), 1)).astype(jnp.float32)
  o_ref[...] = jnp.maximum(
      jnp.dot(oh, emb_ref[...], preferred_element_type=jnp.float32), 0.0)


def _tc_embed(x, emb):
  return pl.pallas_call(
      _emb_body,
      out_shape=jax.ShapeDtypeStruct((N, H), jnp.float32),
  )(x, emb)


def _final_body(h_ref, x_ref, wo1_ref, bo1_ref, wo2_ref, bo2_ref, o_ref):
  h = h_ref[...]
  t = jnp.maximum(
      jnp.dot(h, wo1_ref[...], preferred_element_type=jnp.float32)
      + bo1_ref[...], 0.0)
  o = jnp.dot(t, wo2_ref[...], preferred_element_type=jnp.float32) + bo2_ref[...]
  a = x_ref[...]
  oh = (a == lax.broadcasted_iota(jnp.int32, (N, N_ATOM), 1)).astype(jnp.float32)
  feat = jnp.concatenate([o, oh], axis=1)
  nrm = jnp.sqrt(jnp.sum(feat * feat, axis=1, keepdims=True))
  o_ref[...] = feat / jnp.maximum(nrm, 1e-12)


def _tc_final(h, x, Wo1, bo1, Wo2, bo2):
  return pl.pallas_call(
      _final_body,
      out_shape=jax.ShapeDtypeStruct((N, H + N_ATOM), jnp.float32),
  )(h, x, Wo1, bo1, Wo2, bo2)


def kernel(x, edge_index, edge_attr, emb, W1, b1, W2, b2, W_root, b_conv,
           W_ih, W_hh, b_ih, b_hh, Wo1, bo1, Wo2, bo2):
  src = edge_index[0]
  dst = edge_index[1]
  pad = EPAD - E
  src2 = jnp.pad(src, (0, pad)).reshape(1, EPAD)
  dst2 = jnp.pad(dst, (0, pad)).reshape(1, EPAD)
  ea_pad = jnp.pad(edge_attr, ((0, pad), (0, 0)))
  zeros_tab = jnp.zeros((N, H), jnp.float32)

  b1r = b1.reshape(1, 128)
  W2o = W2.reshape(128, H, H).transpose(0, 2, 1).reshape(128, H * H)
  b2o = b2.reshape(H, H).T.reshape(1, H * H)
  rsum = (jnp.arange(H * H, dtype=jnp.int32)[:, None] // H
          == jnp.arange(H, dtype=jnp.int32)[None, :]).astype(jnp.float32)
  bcr = b_conv.reshape(1, H)
  bihr = b_ih.reshape(1, 3 * H)
  bhhr = b_hh.reshape(1, 3 * H)
  bo1r = bo1.reshape(1, H)
  bo2r = bo2.reshape(1, H)
  W_ihT = W_ih.T
  W_hhT = W_hh.T

  out = _tc_embed(x, emb)
  h = out
  for _ in range(ITERS):
    xs = _sc_gather(out, src2)
    msg = _tc_msg(ea_pad, xs, W1, b1r, W2o, b2o, rsum)
    agg2 = _sc_scatter_add(msg, dst2, zeros_tab)
    h = _tc_node_update(agg2, out, h, W_root, bcr, W_ihT, W_hhT, bihr, bhhr)
    out = h
  return _tc_final(h, x, Wo1, bo1r, Wo2, bo2r)


# speedup vs baseline: 3.6401x; 1.2026x over previous
import functools

import jax
import jax.numpy as jnp
from jax import lax
from jax.experimental import pallas as pl
from jax.experimental.pallas import tpu as pltpu
from jax.experimental.pallas import tpu_sc as plsc

N = 10000
E = 160000
H = 32
N_ATOM = 16
ITERS = 2

GW = 128
NWIN = E // GW
BE = 2000

_NC = 2
_NS = 16


@functools.lru_cache(maxsize=1)
def _vector_mesh():
  return plsc.VectorSubcoreMesh(
      core_axis_name="core", subcore_axis_name="subcore",
      num_cores=_NC, num_subcores=_NS)


def _sc_gather(table, idx2d):
  @functools.partial(
      pl.kernel,
      out_type=jax.ShapeDtypeStruct((E, H), jnp.float32),
      mesh=_vector_mesh(),
      compiler_params=pltpu.CompilerParams(use_tc_tiling_on_sc=False),
  )
  def gk(tab_hbm, i_hbm, o_hbm):
    def body(i_vmem, o_vmem):
      pltpu.sync_copy(tab_hbm.at[i_vmem.at[0]], o_vmem)

    pltpu.emit_pipeline(
        body,
        grid=(NWIN,),
        in_specs=[pl.BlockSpec((1, GW), lambda i: (0, i))],
        out_specs=[pl.BlockSpec((GW, H), lambda i: (i, 0))],
        core_axis_name=("core", "subcore"),
        dimension_semantics=(pltpu.PARALLEL,),
    )(i_hbm, o_hbm)

  return gk(table, idx2d)


def _sc_scatter_add(msg, idx2d, zeros_tab):
  @functools.partial(
      pl.kernel,
      out_type=jax.ShapeDtypeStruct((_NC, N, H), jnp.float32),
      mesh=_vector_mesh(),
      scratch_types=[pltpu.VMEM_SHARED((N, H), jnp.float32)],
      compiler_params=pltpu.CompilerParams(use_tc_tiling_on_sc=False),
  )
  def sk(m_hbm, i_hbm, z_hbm, o_hbm, acc_shared):
    cid = lax.axis_index("core")
    sid = lax.axis_index("subcore")
    rows = N // _NS
    sl = pl.ds(sid * rows, rows)
    pltpu.sync_copy(z_hbm, acc_shared.at[sl])
    plsc.subcore_barrier()

    def body(m_vmem, i_vmem):
      pltpu.sync_copy(m_vmem, acc_shared.at[i_vmem.at[0]], add=True)

    pltpu.emit_pipeline(
        body,
        grid=(NWIN,),
        in_specs=[
            pl.BlockSpec((GW, H), lambda i: (i, 0)),
            pl.BlockSpec((1, GW), lambda i: (0, i)),
        ],
        out_specs=[],
        core_axis_name=("core", "subcore"),
        dimension_semantics=(pltpu.PARALLEL,),
    )(m_hbm, i_hbm)

    plsc.subcore_barrier()
    pltpu.sync_copy(acc_shared.at[sl], o_hbm.at[cid].at[sl])

  return sk(msg, idx2d, zeros_tab)


def _msg_body(ea_ref, xs_ref, w1_ref, b1_ref, w2o_ref, b2o_ref, rsum_ref,
              o_ref):
  eh = jnp.maximum(
      jnp.dot(ea_ref[...], w1_ref[...], preferred_element_type=jnp.float32)
      + b1_ref[...], 0.0)
  we = jnp.dot(eh.astype(jnp.bfloat16), w2o_ref[...],
               preferred_element_type=jnp.float32) + b2o_ref[...]
  xsrep = pltpu.repeat(xs_ref[...], H, axis=1)
  o_ref[...] = jnp.dot((we * xsrep).astype(jnp.bfloat16), rsum_ref[...],
                       preferred_element_type=jnp.float32)


def _tc_msg(ea_pad, xs, W1, b1r, W2o, b2o, rsum):
  return pl.pallas_call(
      _msg_body,
      grid=(E // BE,),
      in_specs=[
          pl.BlockSpec((BE, 4), lambda i: (i, 0)),
          pl.BlockSpec((BE, H), lambda i: (i, 0)),
          pl.BlockSpec((4, 128), lambda i: (0, 0)),
          pl.BlockSpec((1, 128), lambda i: (0, 0)),
          pl.BlockSpec((128, H * H), lambda i: (0, 0)),
          pl.BlockSpec((1, H * H), lambda i: (0, 0)),
          pl.BlockSpec((H * H, H), lambda i: (0, 0)),
      ],
      out_specs=pl.BlockSpec((BE, H), lambda i: (i, 0)),
      out_shape=jax.ShapeDtypeStruct((E, H), jnp.float32),
  )(ea_pad, xs, W1, b1r, W2o, b2o, rsum)


def _node_body(agg2_ref, out_ref, h_ref, wr_ref, bc_ref, wih_ref, whh_ref,
               bih_ref, bhh_ref, o_ref):
  agg = agg2_ref[0] + agg2_ref[1]
  out = out_ref[...]
  h = h_ref[...]
  m = jnp.maximum(
      agg + jnp.dot(out, wr_ref[...], preferred_element_type=jnp.float32)
      + bc_ref[...], 0.0)
  gi = jnp.dot(m, wih_ref[...], preferred_element_type=jnp.float32) + bih_ref[...]
  gh = jnp.dot(h, whh_ref[...], preferred_element_type=jnp.float32) + bhh_ref[...]
  r = jax.nn.sigmoid(gi[:, :H] + gh[:, :H])
  z = jax.nn.sigmoid(gi[:, H:2 * H] + gh[:, H:2 * H])
  n = jnp.tanh(gi[:, 2 * H:] + r * gh[:, 2 * H:])
  o_ref[...] = (1.0 - z) * n + z * h


def _tc_node_update(agg2, out, h, W_root, bc, W_ihT, W_hhT, bih, bhh):
  return pl.pallas_call(
      _node_body,
      out_shape=jax.ShapeDtypeStruct((N, H), jnp.float32),
  )(agg2, out, h, W_root, bc, W_ihT, W_hhT, bih, bhh)


def _emb_body(x_ref, emb_ref, o_ref):
  a = x_ref[...]
  oh = (a == lax.broadcasted_iota(jnp.int32, (N, N_ATOM), 1)).astype(jnp.float32)
  o_ref[...] = jnp.maximum(
      jnp.dot(oh, emb_ref[...], preferred_element_type=jnp.float32), 0.0)


def _tc_embed(x, emb):
  return pl.pallas_call(
      _emb_body,
      out_shape=jax.ShapeDtypeStruct((N, H), jnp.float32),
  )(x, emb)


def _final_body(h_ref, x_ref, wo1_ref, bo1_ref, wo2_ref, bo2_ref, o_ref):
  h = h_ref[...]
  t = jnp.maximum(
      jnp.dot(h, wo1_ref[...], preferred_element_type=jnp.float32)
      + bo1_ref[...], 0.0)
  o = jnp.dot(t, wo2_ref[...], preferred_element_type=jnp.float32) + bo2_ref[...]
  a = x_ref[...]
  oh = (a == lax.broadcasted_iota(jnp.int32, (N, N_ATOM), 1)).astype(jnp.float32)
  feat = jnp.concatenate([o, oh], axis=1)
  nrm = jnp.sqrt(jnp.sum(feat * feat, axis=1, keepdims=True))
  o_ref[...] = feat / jnp.maximum(nrm, 1e-12)


def _tc_final(h, x, Wo1, bo1, Wo2, bo2):
  return pl.pallas_call(
      _final_body,
      out_shape=jax.ShapeDtypeStruct((N, H + N_ATOM), jnp.float32),
  )(h, x, Wo1, bo1, Wo2, bo2)


def kernel(x, edge_index, edge_attr, emb, W1, b1, W2, b2, W_root, b_conv,
           W_ih, W_hh, b_ih, b_hh, Wo1, bo1, Wo2, bo2):
  src2 = edge_index[0].reshape(1, E)
  dst2 = edge_index[1].reshape(1, E)
  zeros_tab = jnp.zeros((N // _NS, H), jnp.float32)

  b1r = b1.reshape(1, 128)
  W2o = W2.reshape(128, H, H).transpose(0, 2, 1).reshape(
      128, H * H).astype(jnp.bfloat16)
  b2o = b2.reshape(H, H).T.reshape(1, H * H)
  rsum = (jnp.arange(H * H, dtype=jnp.int32)[:, None] // H
          == jnp.arange(H, dtype=jnp.int32)[None, :]).astype(jnp.bfloat16)
  bcr = b_conv.reshape(1, H)
  bihr = b_ih.reshape(1, 3 * H)
  bhhr = b_hh.reshape(1, 3 * H)
  bo1r = bo1.reshape(1, H)
  bo2r = bo2.reshape(1, H)
  W_ihT = W_ih.T
  W_hhT = W_hh.T

  out = _tc_embed(x, emb)
  h = out
  for _ in range(ITERS):
    xs = _sc_gather(out, src2)
    msg = _tc_msg(edge_attr, xs, W1, b1r, W2o, b2o, rsum)
    agg2 = _sc_scatter_add(msg, dst2, zeros_tab)
    h = _tc_node_update(agg2, out, h, W_root, bcr, W_ihT, W_hhT, bihr, bhhr)
    out = h
  return _tc_final(h, x, Wo1, bo1r, Wo2, bo2r)


# speedup vs baseline: 3.6539x; 1.0038x over previous
import functools

import jax
import jax.numpy as jnp
from jax import lax
from jax.experimental import pallas as pl
from jax.experimental.pallas import tpu as pltpu
from jax.experimental.pallas import tpu_sc as plsc

N = 10000
E = 160000
H = 32
N_ATOM = 16
ITERS = 2

GW = 128
NWIN = E // GW
BE = 2000

_NC = 2
_NS = 16


@functools.lru_cache(maxsize=1)
def _vector_mesh():
  return plsc.VectorSubcoreMesh(
      core_axis_name="core", subcore_axis_name="subcore",
      num_cores=_NC, num_subcores=_NS)


def _sc_gather(table, idx2d):
  @functools.partial(
      pl.kernel,
      out_type=jax.ShapeDtypeStruct((E, H), jnp.float32),
      mesh=_vector_mesh(),
      compiler_params=pltpu.CompilerParams(use_tc_tiling_on_sc=False),
  )
  def gk(tab_hbm, i_hbm, o_hbm):
    def body(i_vmem, o_vmem):
      pltpu.sync_copy(tab_hbm.at[i_vmem.at[0]], o_vmem)

    pltpu.emit_pipeline(
        body,
        grid=(NWIN,),
        in_specs=[pl.BlockSpec((1, GW), lambda i: (0, i))],
        out_specs=[pl.BlockSpec((GW, H), lambda i: (i, 0))],
        core_axis_name=("core", "subcore"),
        dimension_semantics=(pltpu.PARALLEL,),
    )(i_hbm, o_hbm)

  return gk(table, idx2d)


def _sc_scatter_add(msg, idx2d, zeros_tab):
  @functools.partial(
      pl.kernel,
      out_type=jax.ShapeDtypeStruct((_NC, N, H), jnp.float32),
      mesh=_vector_mesh(),
      scratch_types=[pltpu.VMEM_SHARED((N, H), jnp.float32)],
      compiler_params=pltpu.CompilerParams(use_tc_tiling_on_sc=False),
  )
  def sk(m_hbm, i_hbm, z_hbm, o_hbm, acc_shared):
    cid = lax.axis_index("core")
    sid = lax.axis_index("subcore")
    rows = N // _NS
    sl = pl.ds(sid * rows, rows)
    pltpu.sync_copy(z_hbm, acc_shared.at[sl])
    plsc.subcore_barrier()

    def body(m_vmem, i_vmem):
      pltpu.sync_copy(m_vmem, acc_shared.at[i_vmem.at[0]], add=True)

    pltpu.emit_pipeline(
        body,
        grid=(NWIN,),
        in_specs=[
            pl.BlockSpec((GW, H), lambda i: (i, 0)),
            pl.BlockSpec((1, GW), lambda i: (0, i)),
        ],
        out_specs=[],
        core_axis_name=("core", "subcore"),
        dimension_semantics=(pltpu.PARALLEL,),
    )(m_hbm, i_hbm)

    plsc.subcore_barrier()
    pltpu.sync_copy(acc_shared.at[sl], o_hbm.at[cid].at[sl])

  return sk(msg, idx2d, zeros_tab)


def _msg_body(ea_ref, xs_ref, w1_ref, b1_ref, w2o_ref, rsum_ref, o_ref):
  eh = jnp.maximum(
      jnp.dot(ea_ref[...], w1_ref[...], preferred_element_type=jnp.float32)
      + b1_ref[...], 0.0)
  ehc = jnp.concatenate(
      [eh.astype(jnp.bfloat16),
       jnp.ones((eh.shape[0], 1), jnp.bfloat16)], axis=1)
  we = jnp.dot(ehc, w2o_ref[...],
               preferred_element_type=jnp.float32).astype(jnp.bfloat16)
  xsrep = pltpu.repeat(xs_ref[...].astype(jnp.bfloat16), H, axis=1)
  o_ref[...] = jnp.dot(we * xsrep, rsum_ref[...],
                       preferred_element_type=jnp.float32)


def _tc_msg(ea_pad, xs, W1, b1r, W2o, rsum):
  return pl.pallas_call(
      _msg_body,
      grid=(E // BE,),
      in_specs=[
          pl.BlockSpec((BE, 4), lambda i: (i, 0)),
          pl.BlockSpec((BE, H), lambda i: (i, 0)),
          pl.BlockSpec((4, 128), lambda i: (0, 0)),
          pl.BlockSpec((1, 128), lambda i: (0, 0)),
          pl.BlockSpec((129, H * H), lambda i: (0, 0)),
          pl.BlockSpec((H * H, H), lambda i: (0, 0)),
      ],
      out_specs=pl.BlockSpec((BE, H), lambda i: (i, 0)),
      out_shape=jax.ShapeDtypeStruct((E, H), jnp.float32),
  )(ea_pad, xs, W1, b1r, W2o, rsum)


def _node_body(agg2_ref, out_ref, h_ref, wr_ref, bc_ref, wih_ref, whh_ref,
               bih_ref, bhh_ref, o_ref):
  agg = agg2_ref[0] + agg2_ref[1]
  out = out_ref[...]
  h = h_ref[...]
  m = jnp.maximum(
      agg + jnp.dot(out, wr_ref[...], preferred_element_type=jnp.float32)
      + bc_ref[...], 0.0)
  gi = jnp.dot(m, wih_ref[...], preferred_element_type=jnp.float32) + bih_ref[...]
  gh = jnp.dot(h, whh_ref[...], preferred_element_type=jnp.float32) + bhh_ref[...]
  r = jax.nn.sigmoid(gi[:, :H] + gh[:, :H])
  z = jax.nn.sigmoid(gi[:, H:2 * H] + gh[:, H:2 * H])
  n = jnp.tanh(gi[:, 2 * H:] + r * gh[:, 2 * H:])
  o_ref[...] = (1.0 - z) * n + z * h


def _tc_node_update(agg2, out, h, W_root, bc, W_ihT, W_hhT, bih, bhh):
  return pl.pallas_call(
      _node_body,
      out_shape=jax.ShapeDtypeStruct((N, H), jnp.float32),
  )(agg2, out, h, W_root, bc, W_ihT, W_hhT, bih, bhh)


def _emb_body(x_ref, emb_ref, o_ref):
  a = x_ref[...]
  oh = (a == lax.broadcasted_iota(jnp.int32, (N, N_ATOM), 1)).astype(jnp.float32)
  o_ref[...] = jnp.maximum(
      jnp.dot(oh, emb_ref[...], preferred_element_type=jnp.float32), 0.0)


def _tc_embed(x, emb):
  return pl.pallas_call(
      _emb_body,
      out_shape=jax.ShapeDtypeStruct((N, H), jnp.float32),
  )(x, emb)


def _final_body(h_ref, x_ref, wo1_ref, bo1_ref, wo2_ref, bo2_ref, o_ref):
  h = h_ref[...]
  t = jnp.maximum(
      jnp.dot(h, wo1_ref[...], preferred_element_type=jnp.float32)
      + bo1_ref[...], 0.0)
  o = jnp.dot(t, wo2_ref[...], preferred_element_type=jnp.float32) + bo2_ref[...]
  a = x_ref[...]
  oh = (a == lax.broadcasted_iota(jnp.int32, (N, N_ATOM), 1)).astype(jnp.float32)
  feat = jnp.concatenate([o, oh], axis=1)
  nrm = jnp.sqrt(jnp.sum(feat * feat, axis=1, keepdims=True))
  o_ref[...] = feat / jnp.maximum(nrm, 1e-12)


def _tc_final(h, x, Wo1, bo1, Wo2, bo2):
  return pl.pallas_call(
      _final_body,
      out_shape=jax.ShapeDtypeStruct((N, H + N_ATOM), jnp.float32),
  )(h, x, Wo1, bo1, Wo2, bo2)


def kernel(x, edge_index, edge_attr, emb, W1, b1, W2, b2, W_root, b_conv,
           W_ih, W_hh, b_ih, b_hh, Wo1, bo1, Wo2, bo2):
  src2 = edge_index[0].reshape(1, E)
  dst2 = edge_index[1].reshape(1, E)
  zeros_tab = jnp.zeros((N // _NS, H), jnp.float32)

  b1r = b1.reshape(1, 128)
  W2o = jnp.concatenate([
      W2.reshape(128, H, H).transpose(0, 2, 1).reshape(128, H * H),
      b2.reshape(H, H).T.reshape(1, H * H)], axis=0).astype(jnp.bfloat16)
  rsum = (jnp.arange(H * H, dtype=jnp.int32)[:, None] // H
          == jnp.arange(H, dtype=jnp.int32)[None, :]).astype(jnp.bfloat16)
  bcr = b_conv.reshape(1, H)
  bihr = b_ih.reshape(1, 3 * H)
  bhhr = b_hh.reshape(1, 3 * H)
  bo1r = bo1.reshape(1, H)
  bo2r = bo2.reshape(1, H)
  W_ihT = W_ih.T
  W_hhT = W_hh.T

  out = _tc_embed(x, emb)
  h = out
  for _ in range(ITERS):
    xs = _sc_gather(out, src2)
    msg = _tc_msg(edge_attr, xs, W1, b1r, W2o, rsum)
    agg2 = _sc_scatter_add(msg, dst2, zeros_tab)
    h = _tc_node_update(agg2, out, h, W_root, bcr, W_ihT, W_hhT, bihr, bhhr)
    out = h
  return _tc_final(h, x, Wo1, bo1r, Wo2, bo2r)


# speedup vs baseline: 4.6568x; 1.2745x over previous
import functools

import jax
import jax.numpy as jnp
from jax import lax
from jax.experimental import pallas as pl
from jax.experimental.pallas import tpu as pltpu
from jax.experimental.pallas import tpu_sc as plsc

N = 10000
E = 160000
H = 32
N_ATOM = 16
ITERS = 2

GW = 128
NWIN = E // GW
Q = E // 4
BR = 800

_NC = 2
_NS = 16


@functools.lru_cache(maxsize=1)
def _vector_mesh():
  return plsc.VectorSubcoreMesh(
      core_axis_name="core", subcore_axis_name="subcore",
      num_cores=_NC, num_subcores=_NS)


def _sc_gather(table, idx2d):
  @functools.partial(
      pl.kernel,
      out_type=jax.ShapeDtypeStruct((E, H), jnp.float32),
      mesh=_vector_mesh(),
      compiler_params=pltpu.CompilerParams(use_tc_tiling_on_sc=False),
  )
  def gk(tab_hbm, i_hbm, o_hbm):
    def body(i_vmem, o_vmem):
      pltpu.sync_copy(tab_hbm.at[i_vmem.at[0]], o_vmem)

    pltpu.emit_pipeline(
        body,
        grid=(NWIN,),
        in_specs=[pl.BlockSpec((1, GW), lambda i: (0, i))],
        out_specs=[pl.BlockSpec((GW, H), lambda i: (i, 0))],
        core_axis_name=("core", "subcore"),
        dimension_semantics=(pltpu.PARALLEL,),
    )(i_hbm, o_hbm)

  return gk(table, idx2d)


def _sc_scatter_add(msg, idx2d, zeros_tab):
  @functools.partial(
      pl.kernel,
      out_type=jax.ShapeDtypeStruct((_NC, N, H), jnp.float32),
      mesh=_vector_mesh(),
      scratch_types=[pltpu.VMEM_SHARED((N, H), jnp.float32)],
      compiler_params=pltpu.CompilerParams(use_tc_tiling_on_sc=False),
  )
  def sk(m_hbm, i_hbm, z_hbm, o_hbm, acc_shared):
    cid = lax.axis_index("core")
    sid = lax.axis_index("subcore")
    rows = N // _NS
    sl = pl.ds(sid * rows, rows)
    pltpu.sync_copy(z_hbm, acc_shared.at[sl])
    plsc.subcore_barrier()

    def body(m_vmem, i_vmem):
      pltpu.sync_copy(m_vmem, acc_shared.at[i_vmem.at[0]], add=True)

    pltpu.emit_pipeline(
        body,
        grid=(NWIN,),
        in_specs=[
            pl.BlockSpec((GW, H), lambda i: (i, 0)),
            pl.BlockSpec((1, GW), lambda i: (0, i)),
        ],
        out_specs=[],
        core_axis_name=("core", "subcore"),
        dimension_semantics=(pltpu.PARALLEL,),
    )(m_hbm, i_hbm)

    plsc.subcore_barrier()
    pltpu.sync_copy(acc_shared.at[sl], o_hbm.at[cid].at[sl])

  return sk(msg, idx2d, zeros_tab)


def _msg_body(ea0_ref, ea1_ref, ea2_ref, ea3_ref, xs_ref, w1_ref, b1_ref,
              w2o_ref, rsum_ref, o_ref):
  accs = []
  for q, ea_ref in enumerate((ea0_ref, ea1_ref, ea2_ref, ea3_ref)):
    eh = jnp.maximum(
        jnp.dot(ea_ref[...], w1_ref[...], preferred_element_type=jnp.float32)
        + b1_ref[...], 0.0)
    ehc = jnp.concatenate(
        [eh.astype(jnp.bfloat16),
         jnp.ones((eh.shape[0], 1), jnp.bfloat16)], axis=1)
    we = jnp.dot(ehc, w2o_ref[...],
                 preferred_element_type=jnp.float32).astype(jnp.bfloat16)
    xs_q = xs_ref[:, q * H:(q + 1) * H]
    xsrep = pltpu.repeat(xs_q.astype(jnp.bfloat16), H, axis=1)
    accs.append(jnp.dot(we * xsrep, rsum_ref[...],
                        preferred_element_type=jnp.float32))
  o_ref[...] = jnp.concatenate(accs, axis=1)


def _tc_msg(ea, xs_p, W1, b1r, W2o, rsum):
  def ea_spec(q):
    return pl.BlockSpec((BR, 4), lambda i, q=q: (q * (Q // BR) + i, 0))

  return pl.pallas_call(
      _msg_body,
      grid=(Q // BR,),
      in_specs=[
          ea_spec(0), ea_spec(1), ea_spec(2), ea_spec(3),
          pl.BlockSpec((BR, 128), lambda i: (i, 0)),
          pl.BlockSpec((4, 128), lambda i: (0, 0)),
          pl.BlockSpec((1, 128), lambda i: (0, 0)),
          pl.BlockSpec((129, H * H), lambda i: (0, 0)),
          pl.BlockSpec((H * H, H), lambda i: (0, 0)),
      ],
      out_specs=pl.BlockSpec((BR, 128), lambda i: (i, 0)),
      out_shape=jax.ShapeDtypeStruct((Q, 128), jnp.float32),
  )(ea, ea, ea, ea, xs_p, W1, b1r, W2o, rsum)


def _node_body(agg2_ref, out_ref, h_ref, wr_ref, bc_ref, wih_ref, whh_ref,
               bih_ref, bhh_ref, o_ref):
  agg = agg2_ref[0] + agg2_ref[1]
  out = out_ref[...]
  h = h_ref[...]
  m = jnp.maximum(
      agg + jnp.dot(out, wr_ref[...], preferred_element_type=jnp.float32)
      + bc_ref[...], 0.0)
  gi = jnp.dot(m, wih_ref[...], preferred_element_type=jnp.float32) + bih_ref[...]
  gh = jnp.dot(h, whh_ref[...], preferred_element_type=jnp.float32) + bhh_ref[...]
  r = jax.nn.sigmoid(gi[:, :H] + gh[:, :H])
  z = jax.nn.sigmoid(gi[:, H:2 * H] + gh[:, H:2 * H])
  n = jnp.tanh(gi[:, 2 * H:] + r * gh[:, 2 * H:])
  o_ref[...] = (1.0 - z) * n + z * h


def _tc_node_update(agg2, out, h, W_root, bc, W_ihT, W_hhT, bih, bhh):
  return pl.pallas_call(
      _node_body,
      out_shape=jax.ShapeDtypeStruct((N, H), jnp.float32),
  )(agg2, out, h, W_root, bc, W_ihT, W_hhT, bih, bhh)


def _emb_body(x_ref, emb_ref, o_ref):
  a = x_ref[...]
  oh = (a == lax.broadcasted_iota(jnp.int32, (N, N_ATOM), 1)).astype(jnp.float32)
  o_ref[...] = jnp.maximum(
      jnp.dot(oh, emb_ref[...], preferred_element_type=jnp.float32), 0.0)


def _tc_embed(x, emb):
  return pl.pallas_call(
      _emb_body,
      out_shape=jax.ShapeDtypeStruct((N, H), jnp.float32),
  )(x, emb)


def _final_body(h_ref, x_ref, wo1_ref, bo1_ref, wo2_ref, bo2_ref, o_ref):
  h = h_ref[...]
  t = jnp.maximum(
      jnp.dot(h, wo1_ref[...], preferred_element_type=jnp.float32)
      + bo1_ref[...], 0.0)
  o = jnp.dot(t, wo2_ref[...], preferred_element_type=jnp.float32) + bo2_ref[...]
  a = x_ref[...]
  oh = (a == lax.broadcasted_iota(jnp.int32, (N, N_ATOM), 1)).astype(jnp.float32)
  feat = jnp.concatenate([o, oh], axis=1)
  nrm = jnp.sqrt(jnp.sum(feat * feat, axis=1, keepdims=True))
  o_ref[...] = feat / jnp.maximum(nrm, 1e-12)


def _tc_final(h, x, Wo1, bo1, Wo2, bo2):
  return pl.pallas_call(
      _final_body,
      out_shape=jax.ShapeDtypeStruct((N, H + N_ATOM), jnp.float32),
  )(h, x, Wo1, bo1, Wo2, bo2)


def kernel(x, edge_index, edge_attr, emb, W1, b1, W2, b2, W_root, b_conv,
           W_ih, W_hh, b_ih, b_hh, Wo1, bo1, Wo2, bo2):
  src2 = edge_index[0].reshape(4, Q).T.reshape(1, E)
  dst2 = edge_index[1].reshape(4, Q).T.reshape(1, E)
  zeros_tab = jnp.zeros((N // _NS, H), jnp.float32)

  b1r = b1.reshape(1, 128)
  W2o = jnp.concatenate([
      W2.reshape(128, H, H).transpose(0, 2, 1).reshape(128, H * H),
      b2.reshape(H, H).T.reshape(1, H * H)], axis=0).astype(jnp.bfloat16)
  rsum = (jnp.arange(H * H, dtype=jnp.int32)[:, None] // H
          == jnp.arange(H, dtype=jnp.int32)[None, :]).astype(jnp.bfloat16)
  bcr = b_conv.reshape(1, H)
  bihr = b_ih.reshape(1, 3 * H)
  bhhr = b_hh.reshape(1, 3 * H)
  bo1r = bo1.reshape(1, H)
  bo2r = bo2.reshape(1, H)
  W_ihT = W_ih.T
  W_hhT = W_hh.T

  out = _tc_embed(x, emb)
  h = out
  for _ in range(ITERS):
    xs = _sc_gather(out, src2)
    xs_p = jnp.reshape(xs, (Q, 128))
    msg_p = _tc_msg(edge_attr, xs_p, W1, b1r, W2o, rsum)
    msg = jnp.reshape(msg_p, (E, H))
    agg2 = _sc_scatter_add(msg, dst2, zeros_tab)
    h = _tc_node_update(agg2, out, h, W_root, bcr, W_ihT, W_hhT, bihr, bhhr)
    out = h
  return _tc_final(h, x, Wo1, bo1r, Wo2, bo2r)


# speedup vs baseline: 4.7861x; 1.0278x over previous
import functools

import jax
import jax.numpy as jnp
from jax import lax
from jax.experimental import pallas as pl
from jax.experimental.pallas import tpu as pltpu
from jax.experimental.pallas import tpu_sc as plsc

N = 10000
E = 160000
H = 32
N_ATOM = 16
ITERS = 2

GW = 256
NWIN = E // GW
Q = E // 4
BR = 800

_NC = 2
_NS = 16


@functools.lru_cache(maxsize=1)
def _vector_mesh():
  return plsc.VectorSubcoreMesh(
      core_axis_name="core", subcore_axis_name="subcore",
      num_cores=_NC, num_subcores=_NS)


def _sc_gather(table, idx2d):
  @functools.partial(
      pl.kernel,
      out_type=jax.ShapeDtypeStruct((E, H), jnp.float32),
      mesh=_vector_mesh(),
      compiler_params=pltpu.CompilerParams(use_tc_tiling_on_sc=False),
  )
  def gk(tab_hbm, i_hbm, o_hbm):
    def body(i_vmem, o_vmem):
      pltpu.sync_copy(tab_hbm.at[i_vmem.at[0]], o_vmem)

    pltpu.emit_pipeline(
        body,
        grid=(NWIN,),
        in_specs=[pl.BlockSpec((1, GW), lambda i: (0, i))],
        out_specs=[pl.BlockSpec((GW, H), lambda i: (i, 0))],
        core_axis_name=("core", "subcore"),
        dimension_semantics=(pltpu.PARALLEL,),
    )(i_hbm, o_hbm)

  return gk(table, idx2d)


def _sc_scatter_add(msg, idx2d, zeros_tab):
  @functools.partial(
      pl.kernel,
      out_type=jax.ShapeDtypeStruct((_NC, N, H), jnp.float32),
      mesh=_vector_mesh(),
      scratch_types=[pltpu.VMEM_SHARED((N, H), jnp.float32)],
      compiler_params=pltpu.CompilerParams(use_tc_tiling_on_sc=False),
  )
  def sk(m_hbm, i_hbm, z_hbm, o_hbm, acc_shared):
    cid = lax.axis_index("core")
    sid = lax.axis_index("subcore")
    rows = N // _NS
    sl = pl.ds(sid * rows, rows)
    pltpu.sync_copy(z_hbm, acc_shared.at[sl])
    plsc.subcore_barrier()

    def body(m_vmem, i_vmem):
      pltpu.sync_copy(m_vmem, acc_shared.at[i_vmem.at[0]], add=True)

    pltpu.emit_pipeline(
        body,
        grid=(NWIN,),
        in_specs=[
            pl.BlockSpec((GW, H), lambda i: (i, 0)),
            pl.BlockSpec((1, GW), lambda i: (0, i)),
        ],
        out_specs=[],
        core_axis_name=("core", "subcore"),
        dimension_semantics=(pltpu.PARALLEL,),
    )(m_hbm, i_hbm)

    plsc.subcore_barrier()
    pltpu.sync_copy(acc_shared.at[sl], o_hbm.at[cid].at[sl])

  return sk(msg, idx2d, zeros_tab)


def _msg_body(ea0_ref, ea1_ref, ea2_ref, ea3_ref, xs_ref, w1_ref, b1_ref,
              w2o_ref, rsum_ref, o_ref):
  accs = []
  for q, ea_ref in enumerate((ea0_ref, ea1_ref, ea2_ref, ea3_ref)):
    eh = jnp.maximum(
        jnp.dot(ea_ref[...], w1_ref[...], preferred_element_type=jnp.float32)
        + b1_ref[...], 0.0)
    ehc = jnp.concatenate(
        [eh.astype(jnp.bfloat16),
         jnp.ones((eh.shape[0], 1), jnp.bfloat16)], axis=1)
    we = jnp.dot(ehc, w2o_ref[...],
                 preferred_element_type=jnp.float32).astype(jnp.bfloat16)
    xs_q = xs_ref[:, q * H:(q + 1) * H]
    xsrep = pltpu.repeat(xs_q.astype(jnp.bfloat16), H, axis=1)
    accs.append(jnp.dot(we * xsrep, rsum_ref[...],
                        preferred_element_type=jnp.float32))
  o_ref[...] = jnp.concatenate(accs, axis=1)


def _tc_msg(ea, xs_p, W1, b1r, W2o, rsum):
  def ea_spec(q):
    return pl.BlockSpec((BR, 4), lambda i, q=q: (q * (Q // BR) + i, 0))

  return pl.pallas_call(
      _msg_body,
      grid=(Q // BR,),
      in_specs=[
          ea_spec(0), ea_spec(1), ea_spec(2), ea_spec(3),
          pl.BlockSpec((BR, 128), lambda i: (i, 0)),
          pl.BlockSpec((4, 128), lambda i: (0, 0)),
          pl.BlockSpec((1, 128), lambda i: (0, 0)),
          pl.BlockSpec((129, H * H), lambda i: (0, 0)),
          pl.BlockSpec((H * H, H), lambda i: (0, 0)),
      ],
      out_specs=pl.BlockSpec((BR, 128), lambda i: (i, 0)),
      out_shape=jax.ShapeDtypeStruct((Q, 128), jnp.float32),
  )(ea, ea, ea, ea, xs_p, W1, b1r, W2o, rsum)


def _node_body(agg2_ref, out_ref, h_ref, wr_ref, bc_ref, wih_ref, whh_ref,
               bih_ref, bhh_ref, o_ref):
  agg = agg2_ref[0] + agg2_ref[1]
  out = out_ref[...]
  h = h_ref[...]
  m = jnp.maximum(
      agg + jnp.dot(out, wr_ref[...], preferred_element_type=jnp.float32)
      + bc_ref[...], 0.0)
  gi = jnp.dot(m, wih_ref[...], preferred_element_type=jnp.float32) + bih_ref[...]
  gh = jnp.dot(h, whh_ref[...], preferred_element_type=jnp.float32) + bhh_ref[...]
  r = jax.nn.sigmoid(gi[:, :H] + gh[:, :H])
  z = jax.nn.sigmoid(gi[:, H:2 * H] + gh[:, H:2 * H])
  n = jnp.tanh(gi[:, 2 * H:] + r * gh[:, 2 * H:])
  o_ref[...] = (1.0 - z) * n + z * h


def _tc_node_update(agg2, out, h, W_root, bc, W_ihT, W_hhT, bih, bhh):
  return pl.pallas_call(
      _node_body,
      out_shape=jax.ShapeDtypeStruct((N, H), jnp.float32),
  )(agg2, out, h, W_root, bc, W_ihT, W_hhT, bih, bhh)


def _emb_body(x_ref, emb_ref, o_ref):
  a = x_ref[...]
  oh = (a == lax.broadcasted_iota(jnp.int32, (N, N_ATOM), 1)).astype(jnp.float32)
  o_ref[...] = jnp.maximum(
      jnp.dot(oh, emb_ref[...], preferred_element_type=jnp.float32), 0.0)


def _tc_embed(x, emb):
  return pl.pallas_call(
      _emb_body,
      out_shape=jax.ShapeDtypeStruct((N, H), jnp.float32),
  )(x, emb)


def _final_body(h_ref, x_ref, wo1_ref, bo1_ref, wo2_ref, bo2_ref, o_ref):
  h = h_ref[...]
  t = jnp.maximum(
      jnp.dot(h, wo1_ref[...], preferred_element_type=jnp.float32)
      + bo1_ref[...], 0.0)
  o = jnp.dot(t, wo2_ref[...], preferred_element_type=jnp.float32) + bo2_ref[...]
  a = x_ref[...]
  oh = (a == lax.broadcasted_iota(jnp.int32, (N, N_ATOM), 1)).astype(jnp.float32)
  feat = jnp.concatenate([o, oh], axis=1)
  nrm = jnp.sqrt(jnp.sum(feat * feat, axis=1, keepdims=True))
  o_ref[...] = feat / jnp.maximum(nrm, 1e-12)


def _tc_final(h, x, Wo1, bo1, Wo2, bo2):
  return pl.pallas_call(
      _final_body,
      out_shape=jax.ShapeDtypeStruct((N, H + N_ATOM), jnp.float32),
  )(h, x, Wo1, bo1, Wo2, bo2)


def kernel(x, edge_index, edge_attr, emb, W1, b1, W2, b2, W_root, b_conv,
           W_ih, W_hh, b_ih, b_hh, Wo1, bo1, Wo2, bo2):
  src2 = edge_index[0].reshape(4, Q).T.reshape(1, E)
  dst2 = edge_index[1].reshape(4, Q).T.reshape(1, E)
  zeros_tab = jnp.zeros((N // _NS, H), jnp.float32)

  b1r = b1.reshape(1, 128)
  W2o = jnp.concatenate([
      W2.reshape(128, H, H).transpose(0, 2, 1).reshape(128, H * H),
      b2.reshape(H, H).T.reshape(1, H * H)], axis=0).astype(jnp.bfloat16)
  rsum = (jnp.arange(H * H, dtype=jnp.int32)[:, None] // H
          == jnp.arange(H, dtype=jnp.int32)[None, :]).astype(jnp.bfloat16)
  bcr = b_conv.reshape(1, H)
  bihr = b_ih.reshape(1, 3 * H)
  bhhr = b_hh.reshape(1, 3 * H)
  bo1r = bo1.reshape(1, H)
  bo2r = bo2.reshape(1, H)
  W_ihT = W_ih.T
  W_hhT = W_hh.T

  out = _tc_embed(x, emb)
  h = out
  for _ in range(ITERS):
    xs = _sc_gather(out, src2)
    xs_p = jnp.reshape(xs, (Q, 128))
    msg_p = _tc_msg(edge_attr, xs_p, W1, b1r, W2o, rsum)
    msg = jnp.reshape(msg_p, (E, H))
    agg2 = _sc_scatter_add(msg, dst2, zeros_tab)
    h = _tc_node_update(agg2, out, h, W_root, bcr, W_ihT, W_hhT, bihr, bhhr)
    out = h
  return _tc_final(h, x, Wo1, bo1r, Wo2, bo2r)
